# Initial kernel scaffold; baseline (speedup 1.0000x reference)
import jax
import jax.numpy as jnp
from jax.experimental import pallas as pl


def kernel(x, emb, W1, b1, W2, b2):
    raise NotImplementedError("write your pallas kernel here")


# speedup vs baseline: 2.1681x; 2.1681x over previous
import functools

import jax
import jax.numpy as jnp
from jax import lax
from jax.experimental import pallas as pl
from jax.experimental.pallas import tpu as pltpu
from jax.experimental.pallas import tpu_sc as plsc

_B = 4096
_HIST = 200
_D = 128
_H = 64
_O = 2
_L = 16
_NC = 2
_NS = 16
_NW = _NC * _NS
_BPW = _B // _NW
_C0 = 104
_C1 = _HIST - _C0

_mesh = plsc.VectorSubcoreMesh(core_axis_name="c", subcore_axis_name="s")


@functools.partial(
    pl.kernel,
    mesh=_mesh,
    out_type=jax.ShapeDtypeStruct((_B, _D), jnp.float32),
    scratch_types=[
        pltpu.VMEM((_BPW * _HIST,), jnp.int32),
        pltpu.VMEM((_HIST, _D), jnp.float32),
        pltpu.VMEM((_HIST, _D), jnp.float32),
        pltpu.VMEM((_BPW, _D), jnp.float32),
        pltpu.SemaphoreType.DMA,
        pltpu.SemaphoreType.DMA,
    ],
)
def _pool_sum(x_hbm, emb_hbm, out_hbm, idx_v, buf0, buf1, outb, sem0, sem1):
    wid = lax.axis_index("s") * _NC + lax.axis_index("c")
    base = pl.multiple_of(wid * _BPW, _BPW)
    pltpu.sync_copy(x_hbm.at[pl.ds(base * _HIST, _BPW * _HIST)], idx_v)

    bufs = (buf0, buf1)
    sems = (sem0, sem1)

    def copies(b, k):
        off = pl.multiple_of(b * _HIST, 8)
        off1 = pl.multiple_of(b * _HIST + _C0, 8)
        return (
            pltpu.make_async_copy(
                emb_hbm.at[idx_v.at[pl.ds(off, _C0)]],
                bufs[k].at[pl.ds(0, _C0)],
                sems[k],
            ),
            pltpu.make_async_copy(
                emb_hbm.at[idx_v.at[pl.ds(off1, _C1)]],
                bufs[k].at[pl.ds(_C0, _C1)],
                sems[k],
            ),
        )

    def start_gather(b, k):
        for cp in copies(b, k):
            cp.start()

    def wait_gather(b, k):
        for cp in copies(b, k):
            cp.wait()

    start_gather(0, 0)

    def body(b2, carry_unused):
        for k in range(2):
            b = b2 * 2 + k

            @pl.when(b + 1 < _BPW)
            def _():
                start_gather(b + 1, (k + 1) % 2)

            wait_gather(b, k)
            buf = bufs[k]

            def acc_body(j, acc):
                return tuple(
                    acc[i] + buf[j, pl.ds(i * _L, _L)] for i in range(_D // _L)
                )

            zeros = tuple(
                jnp.zeros((_L,), jnp.float32) for _ in range(_D // _L)
            )
            acc = lax.fori_loop(0, _HIST, acc_body, zeros)
            for i in range(_D // _L):
                outb[b, pl.ds(i * _L, _L)] = acc[i]
        return carry_unused

    lax.fori_loop(0, _BPW // 2, body, 0)
    pltpu.sync_copy(outb, out_hbm.at[pl.ds(base, _BPW)])


def _mlp_body(pooled_ref, w1_ref, b1_ref, w2_ref, b2_ref, out_ref):
    pooled = pooled_ref[...] * (1.0 / _HIST)
    h = lax.dot_general(
        pooled, w1_ref[...], (((1,), (1,)), ((), ())),
        preferred_element_type=jnp.float32,
    ) + b1_ref[...]
    h = jnp.maximum(h, 0.0)
    out_ref[...] = lax.dot_general(
        h, w2_ref[...], (((1,), (1,)), ((), ())),
        preferred_element_type=jnp.float32,
    ) + b2_ref[...]


def kernel(x, emb, W1, b1, W2, b2):
    pooled_sum = _pool_sum(x.reshape(-1), emb)
    return pl.pallas_call(
        _mlp_body,
        out_shape=jax.ShapeDtypeStruct((_B, _O), jnp.float32),
    )(pooled_sum, W1, b1.reshape(1, _H), W2, b2.reshape(1, _O))


# speedup vs baseline: 2.1728x; 1.0022x over previous
import functools

import jax
import jax.numpy as jnp
from jax import lax
from jax.experimental import pallas as pl
from jax.experimental.pallas import tpu as pltpu
from jax.experimental.pallas import tpu_sc as plsc

_B = 4096
_HIST = 200
_D = 128
_H = 64
_O = 2
_L = 16
_NC = 2
_NS = 16
_NW = _NC * _NS
_BPW = _B // _NW
_C0 = 104
_C1 = _HIST - _C0
_UNROLL = 4

_mesh = plsc.VectorSubcoreMesh(core_axis_name="c", subcore_axis_name="s")


@functools.partial(
    pl.kernel,
    mesh=_mesh,
    out_type=jax.ShapeDtypeStruct((_B, _D), jnp.float32),
    scratch_types=[
        pltpu.VMEM((_BPW * _HIST,), jnp.int32),
        pltpu.VMEM((_HIST, _D), jnp.float32),
        pltpu.VMEM((_HIST, _D), jnp.float32),
        pltpu.VMEM((_BPW, _D), jnp.float32),
        pltpu.SemaphoreType.DMA,
        pltpu.SemaphoreType.DMA,
    ],
)
def _pool_sum(x_hbm, emb_hbm, out_hbm, idx_v, buf0, buf1, outb, sem0, sem1):
    wid = lax.axis_index("s") * _NC + lax.axis_index("c")
    base = pl.multiple_of(wid * _BPW, _BPW)
    pltpu.sync_copy(x_hbm.at[pl.ds(base * _HIST, _BPW * _HIST)], idx_v)

    bufs = (buf0, buf1)
    sems = (sem0, sem1)

    def copies(b, k):
        off = pl.multiple_of(b * _HIST, 8)
        off1 = pl.multiple_of(b * _HIST + _C0, 8)
        return (
            pltpu.make_async_copy(
                emb_hbm.at[idx_v.at[pl.ds(off, _C0)]],
                bufs[k].at[pl.ds(0, _C0)],
                sems[k],
            ),
            pltpu.make_async_copy(
                emb_hbm.at[idx_v.at[pl.ds(off1, _C1)]],
                bufs[k].at[pl.ds(_C0, _C1)],
                sems[k],
            ),
        )

    def start_gather(b, k):
        for cp in copies(b, k):
            cp.start()

    def wait_gather(b, k):
        for cp in copies(b, k):
            cp.wait()

    start_gather(0, 0)

    def body(b2, carry_unused):
        for k in range(2):
            b = b2 * 2 + k

            @pl.when(b + 1 < _BPW)
            def _():
                start_gather(b + 1, (k + 1) % 2)

            wait_gather(b, k)
            buf = bufs[k]

            def acc_body(j4, acc):
                j = j4 * _UNROLL
                for u in range(_UNROLL):
                    acc = tuple(
                        acc[i] + buf[j + u, pl.ds(i * _L, _L)]
                        for i in range(_D // _L)
                    )
                return acc

            zeros = tuple(
                jnp.zeros((_L,), jnp.float32) for _ in range(_D // _L)
            )
            acc = lax.fori_loop(0, _HIST // _UNROLL, acc_body, zeros)
            for i in range(_D // _L):
                outb[b, pl.ds(i * _L, _L)] = acc[i]
        return carry_unused

    lax.fori_loop(0, _BPW // 2, body, 0)
    pltpu.sync_copy(outb, out_hbm.at[pl.ds(base, _BPW)])


def _mlp_body(pooled_ref, w1_ref, b1_ref, w2_ref, b2_ref, out_ref):
    pooled = pooled_ref[...] * (1.0 / _HIST)
    h = lax.dot_general(
        pooled, w1_ref[...], (((1,), (1,)), ((), ())),
        preferred_element_type=jnp.float32,
    ) + b1_ref[...]
    h = jnp.maximum(h, 0.0)
    out_ref[...] = lax.dot_general(
        h, w2_ref[...], (((1,), (1,)), ((), ())),
        preferred_element_type=jnp.float32,
    ) + b2_ref[...]


def kernel(x, emb, W1, b1, W2, b2):
    pooled_sum = _pool_sum(x.reshape(-1), emb)
    return pl.pallas_call(
        _mlp_body,
        out_shape=jax.ShapeDtypeStruct((_B, _O), jnp.float32),
    )(pooled_sum, W1, b1.reshape(1, _H), W2, b2.reshape(1, _O))


# speedup vs baseline: 2.6569x; 1.2228x over previous
import functools

import jax
import jax.numpy as jnp
from jax import lax
from jax.experimental import pallas as pl
from jax.experimental.pallas import tpu as pltpu
from jax.experimental.pallas import tpu_sc as plsc

_B = 4096
_HIST = 200
_D = 128
_H = 64
_O = 2
_L = 16
_NC = 2
_NS = 16
_NW = _NC * _NS
_BPW = _B // _NW
_C0 = 104
_C1 = _HIST - _C0
_UNROLL = 4

_mesh = plsc.VectorSubcoreMesh(core_axis_name="c", subcore_axis_name="s")


@functools.partial(
    pl.kernel,
    mesh=_mesh,
    out_type=jax.ShapeDtypeStruct((_B, _D), jnp.float32),
    scratch_types=[
        pltpu.VMEM((_BPW * _HIST,), jnp.int32),
        pltpu.VMEM((_HIST, _D), jnp.float32),
        pltpu.VMEM((_HIST, _D), jnp.float32),
        pltpu.VMEM((_HIST, _D), jnp.float32),
        pltpu.VMEM((_BPW, _D), jnp.float32),
        pltpu.SemaphoreType.DMA,
        pltpu.SemaphoreType.DMA,
        pltpu.SemaphoreType.DMA,
    ],
)
def _pool_sum(x_hbm, emb_hbm, out_hbm, idx_v, buf0, buf1, buf2, outb,
              sem0, sem1, sem2):
    wid = lax.axis_index("s") * _NC + lax.axis_index("c")
    base = pl.multiple_of(wid * _BPW, _BPW)
    pltpu.sync_copy(x_hbm.at[pl.ds(base * _HIST, _BPW * _HIST)], idx_v)

    bufs = (buf0, buf1, buf2)
    sems = (sem0, sem1, sem2)

    def copies(b, k):
        off = pl.multiple_of(b * _HIST, 8)
        off1 = pl.multiple_of(b * _HIST + _C0, 8)
        return (
            pltpu.make_async_copy(
                emb_hbm.at[idx_v.at[pl.ds(off, _C0)]],
                bufs[k].at[pl.ds(0, _C0)],
                sems[k],
            ),
            pltpu.make_async_copy(
                emb_hbm.at[idx_v.at[pl.ds(off1, _C1)]],
                bufs[k].at[pl.ds(_C0, _C1)],
                sems[k],
            ),
        )

    def start_gather(b, k):
        for cp in copies(b, k):
            cp.start()

    def wait_gather(b, k):
        for cp in copies(b, k):
            cp.wait()

    def process(b, k):
        wait_gather(b, k)
        buf = bufs[k]

        def acc_body(j4, acc):
            j = j4 * _UNROLL
            for u in range(_UNROLL):
                acc = tuple(
                    acc[i] + buf[j + u, pl.ds(i * _L, _L)]
                    for i in range(_D // _L)
                )
            return acc

        zeros = tuple(jnp.zeros((_L,), jnp.float32) for _ in range(_D // _L))
        acc = lax.fori_loop(0, _HIST // _UNROLL, acc_body, zeros)
        for i in range(_D // _L):
            outb[b, pl.ds(i * _L, _L)] = acc[i]

    start_gather(0, 0)
    start_gather(1, 1)

    def body(b3, carry_unused):
        for k in range(3):
            b = b3 * 3 + k
            start_gather(b + 2, (k + 2) % 3)
            process(b, k)
        return carry_unused

    _MAIN = (_BPW - 2) // 3
    lax.fori_loop(0, _MAIN, body, 0)
    process(_BPW - 2, (_BPW - 2) % 3)
    process(_BPW - 1, (_BPW - 1) % 3)
    pltpu.sync_copy(outb, out_hbm.at[pl.ds(base, _BPW)])


def _mlp_body(pooled_ref, w1_ref, b1_ref, w2_ref, b2_ref, out_ref):
    pooled = pooled_ref[...] * (1.0 / _HIST)
    h = lax.dot_general(
        pooled, w1_ref[...], (((1,), (1,)), ((), ())),
        preferred_element_type=jnp.float32,
    ) + b1_ref[...]
    h = jnp.maximum(h, 0.0)
    out_ref[...] = lax.dot_general(
        h, w2_ref[...], (((1,), (1,)), ((), ())),
        preferred_element_type=jnp.float32,
    ) + b2_ref[...]


def kernel(x, emb, W1, b1, W2, b2):
    pooled_sum = _pool_sum(x.reshape(-1), emb)
    return pl.pallas_call(
        _mlp_body,
        out_shape=jax.ShapeDtypeStruct((_B, _O), jnp.float32),
    )(pooled_sum, W1, b1.reshape(1, _H), W2, b2.reshape(1, _O))
